# Initial kernel scaffold; baseline (speedup 1.0000x reference)
import functools

import jax
import jax.numpy as jnp
from jax import lax
from jax.experimental import pallas as pl
from jax.experimental.pallas import tpu as pltpu
from jax.experimental.pallas import tpu_sc as plsc

N_NODES = 10000
N_EDGES = 320000
D = 128

NC = 2
NS = 16
NW = NC * NS

EB = 125
EROWS = N_EDGES // EB
ROWS_PER_W = EROWS // NW
NODES_PER_S = N_NODES // NS


def _deg_body(dst_hbm, ones_hbm, zeros_hbm, degp_hbm, dstv, onesv, deg_sh):
    c = lax.axis_index("c")
    s = lax.axis_index("s")
    gid = c * NS + s
    row0 = gid * ROWS_PER_W
    pltpu.sync_copy(dst_hbm.at[pl.ds(row0, ROWS_PER_W)], dstv)
    pltpu.sync_copy(ones_hbm, onesv)
    n0 = s * NODES_PER_S
    pltpu.sync_copy(zeros_hbm.at[pl.ds(n0, NODES_PER_S)],
                    deg_sh.at[pl.ds(n0, NODES_PER_S)])
    plsc.subcore_barrier()

    def step(i, carry):
        pltpu.sync_copy(onesv, deg_sh.at[dstv.at[i]], add=True)
        return carry

    lax.fori_loop(0, ROWS_PER_W, step, 0)
    plsc.subcore_barrier()
    pltpu.sync_copy(deg_sh.at[pl.ds(n0, NODES_PER_S)],
                    degp_hbm.at[c, pl.ds(n0, NODES_PER_S)])


_deg_kernel = functools.partial(
    pl.kernel,
    out_type=jax.ShapeDtypeStruct((NC, N_NODES, 16), jnp.float32),
    mesh=plsc.VectorSubcoreMesh(core_axis_name="c", subcore_axis_name="s"),
    scratch_types=[
        pltpu.VMEM((ROWS_PER_W, EB), jnp.int32),
        pltpu.VMEM((EB, 16), jnp.float32),
        pltpu.VMEM_SHARED((N_NODES, 16), jnp.float32),
    ],
)(_deg_body)


def _agg_body(src_hbm, dst_hbm, y_hbm, zeros_hbm, aggp_hbm,
              srcv, dstv, rows, sem, agg_sh):
    c = lax.axis_index("c")
    s = lax.axis_index("s")
    gid = c * NS + s
    row0 = gid * ROWS_PER_W
    pltpu.sync_copy(src_hbm.at[pl.ds(row0, ROWS_PER_W)], srcv)
    pltpu.sync_copy(dst_hbm.at[pl.ds(row0, ROWS_PER_W)], dstv)
    n0 = s * NODES_PER_S
    pltpu.sync_copy(zeros_hbm.at[pl.ds(n0, NODES_PER_S)],
                    agg_sh.at[pl.ds(n0, NODES_PER_S)])
    plsc.subcore_barrier()

    def step(i, carry):
        pltpu.async_copy(y_hbm.at[srcv.at[i]], rows, sem).wait()
        pltpu.sync_copy(rows, agg_sh.at[dstv.at[i]], add=True)
        return carry

    lax.fori_loop(0, ROWS_PER_W, step, 0)
    plsc.subcore_barrier()
    pltpu.sync_copy(agg_sh.at[pl.ds(n0, NODES_PER_S)],
                    aggp_hbm.at[c, pl.ds(n0, NODES_PER_S)])


_agg_kernel = functools.partial(
    pl.kernel,
    out_type=jax.ShapeDtypeStruct((NC, N_NODES, D), jnp.float32),
    mesh=plsc.VectorSubcoreMesh(core_axis_name="c", subcore_axis_name="s"),
    scratch_types=[
        pltpu.VMEM((ROWS_PER_W, EB), jnp.int32),
        pltpu.VMEM((ROWS_PER_W, EB), jnp.int32),
        pltpu.VMEM((EB, D), jnp.float32),
        pltpu.SemaphoreType.DMA,
        pltpu.VMEM_SHARED((N_NODES, D), jnp.float32),
    ],
)(_agg_body)


BN = 500
NBLK = N_NODES // BN


def _mm_body(degp_ref, x_ref, w_ref, y_ref):
    deg = degp_ref[0, :, 0:1] + degp_ref[1, :, 0:1] + 1.0
    dis = lax.rsqrt(deg)
    xw = jnp.dot(x_ref[...], w_ref[...], preferred_element_type=jnp.float32)
    y_ref[...] = xw * dis


def _mm(degp, x, W):
    return pl.pallas_call(
        _mm_body,
        grid=(NBLK,),
        in_specs=[
            pl.BlockSpec((NC, BN, 16), lambda i: (0, i, 0)),
            pl.BlockSpec((BN, D), lambda i: (i, 0)),
            pl.BlockSpec((D, D), lambda i: (0, 0)),
        ],
        out_specs=pl.BlockSpec((BN, D), lambda i: (i, 0)),
        out_shape=jax.ShapeDtypeStruct((N_NODES, D), jnp.float32),
    )(degp, x, W)


def _final_body(degp_ref, aggp_ref, y_ref, b_ref, out_ref):
    deg = degp_ref[0, :, 0:1] + degp_ref[1, :, 0:1] + 1.0
    dis = lax.rsqrt(deg)
    acc = aggp_ref[0] + aggp_ref[1] + y_ref[...]
    out_ref[...] = acc * dis + b_ref[...]


def _final(degp, aggp, y, b2):
    return pl.pallas_call(
        _final_body,
        grid=(NBLK,),
        in_specs=[
            pl.BlockSpec((NC, BN, 16), lambda i: (0, i, 0)),
            pl.BlockSpec((NC, BN, D), lambda i: (0, i, 0)),
            pl.BlockSpec((BN, D), lambda i: (i, 0)),
            pl.BlockSpec((1, D), lambda i: (0, 0)),
        ],
        out_specs=pl.BlockSpec((BN, D), lambda i: (i, 0)),
        out_shape=jax.ShapeDtypeStruct((N_NODES, D), jnp.float32),
    )(degp, aggp, y, b2)


def kernel(x, edge_index, W, b):
    ei = edge_index.astype(jnp.int32)
    src = ei[0].reshape(EROWS, EB)
    dst = ei[1].reshape(EROWS, EB)
    ones16 = jnp.ones((EB, 16), jnp.float32)
    zeros16 = jnp.zeros((N_NODES, 16), jnp.float32)
    zerosD = jnp.zeros((N_NODES, D), jnp.float32)

    degp = _deg_kernel(dst, ones16, zeros16)
    y = _mm(degp, x, W)
    aggp = _agg_kernel(src, dst, y, zerosD)
    out = _final(degp, aggp, y, b.reshape(1, D))
    return out


# speedup vs baseline: 13.4907x; 13.4907x over previous
import functools

import jax
import jax.numpy as jnp
from jax import lax
from jax.experimental import pallas as pl
from jax.experimental.pallas import tpu as pltpu
from jax.experimental.pallas import tpu_sc as plsc

N_NODES = 10000
N_EDGES = 320000
D = 128

NC = 2
NS = 16
NW = NC * NS

EB = 128
EROWS = 2560
E_PAD = EROWS * EB
ROWS_PER_W = EROWS // NW

N_PAD = 10240
NCH = N_PAD // NS


def _deg_body(dst_hbm, ones_hbm, zeros_hbm, degp_hbm, dstv, onesv, deg_sh):
    c = lax.axis_index("c")
    s = lax.axis_index("s")
    gid = c * NS + s
    row0 = gid * ROWS_PER_W
    pltpu.sync_copy(dst_hbm.at[pl.ds(row0, ROWS_PER_W)], dstv)
    pltpu.sync_copy(ones_hbm, onesv)
    n0 = s * NCH
    pltpu.sync_copy(zeros_hbm.at[pl.ds(n0, NCH)], deg_sh.at[pl.ds(n0, NCH)])
    plsc.subcore_barrier()

    def step(i, carry):
        pltpu.sync_copy(onesv, deg_sh.at[dstv.at[i]], add=True)
        return carry

    lax.fori_loop(0, ROWS_PER_W, step, 0)
    plsc.subcore_barrier()
    pltpu.sync_copy(deg_sh.at[pl.ds(n0, NCH)],
                    degp_hbm.at[c, pl.ds(n0, NCH)])


@functools.cache
def _deg_kernel():
    return pl.kernel(
        _deg_body,
        out_type=jax.ShapeDtypeStruct((NC, N_PAD, D), jnp.float32),
        mesh=plsc.VectorSubcoreMesh(core_axis_name="c", subcore_axis_name="s",
                                    num_cores=NC, num_subcores=NS),
        scratch_types=[
            pltpu.VMEM((ROWS_PER_W, EB), jnp.int32),
            pltpu.VMEM((EB, D), jnp.float32),
            pltpu.VMEM_SHARED((N_PAD, D), jnp.float32),
        ],
    )


def _agg_body(src_hbm, dst_hbm, y_hbm, zeros_hbm, aggp_hbm,
              srcv, dstv, rows, sem, agg_sh):
    c = lax.axis_index("c")
    s = lax.axis_index("s")
    gid = c * NS + s
    row0 = gid * ROWS_PER_W
    pltpu.sync_copy(src_hbm.at[pl.ds(row0, ROWS_PER_W)], srcv)
    pltpu.sync_copy(dst_hbm.at[pl.ds(row0, ROWS_PER_W)], dstv)
    n0 = s * NCH
    pltpu.sync_copy(zeros_hbm.at[pl.ds(n0, NCH)], agg_sh.at[pl.ds(n0, NCH)])
    plsc.subcore_barrier()

    def step(i, carry):
        pltpu.async_copy(y_hbm.at[srcv.at[i]], rows, sem).wait()
        pltpu.sync_copy(rows, agg_sh.at[dstv.at[i]], add=True)
        return carry

    lax.fori_loop(0, ROWS_PER_W, step, 0)
    plsc.subcore_barrier()
    pltpu.sync_copy(agg_sh.at[pl.ds(n0, NCH)],
                    aggp_hbm.at[c, pl.ds(n0, NCH)])


@functools.cache
def _agg_kernel():
    return pl.kernel(
        _agg_body,
        out_type=jax.ShapeDtypeStruct((NC, N_PAD, D), jnp.float32),
        mesh=plsc.VectorSubcoreMesh(core_axis_name="c", subcore_axis_name="s",
                                    num_cores=NC, num_subcores=NS),
        scratch_types=[
            pltpu.VMEM((ROWS_PER_W, EB), jnp.int32),
            pltpu.VMEM((ROWS_PER_W, EB), jnp.int32),
            pltpu.VMEM((EB, D), jnp.float32),
            pltpu.SemaphoreType.DMA,
            pltpu.VMEM_SHARED((N_PAD, D), jnp.float32),
        ],
    )


BM = 1024
BF = 1000


def _mm_body(degp_ref, x_ref, w_ref, y_ref):
    deg = degp_ref[0, :, 0:1] + degp_ref[1, :, 0:1] + 1.0
    dis = lax.rsqrt(deg)
    xw = jnp.dot(x_ref[...], w_ref[...], preferred_element_type=jnp.float32)
    y_ref[...] = xw * dis


def _mm(degp, xp, W):
    return pl.pallas_call(
        _mm_body,
        grid=(N_PAD // BM,),
        in_specs=[
            pl.BlockSpec((NC, BM, D), lambda i: (0, i, 0)),
            pl.BlockSpec((BM, D), lambda i: (i, 0)),
            pl.BlockSpec((D, D), lambda i: (0, 0)),
        ],
        out_specs=pl.BlockSpec((BM, D), lambda i: (i, 0)),
        out_shape=jax.ShapeDtypeStruct((N_PAD, D), jnp.float32),
    )(degp, xp, W)


def _final_body(degp_ref, aggp_ref, y_ref, b_ref, out_ref):
    deg = degp_ref[0, :, 0:1] + degp_ref[1, :, 0:1] + 1.0
    dis = lax.rsqrt(deg)
    acc = aggp_ref[0] + aggp_ref[1] + y_ref[...]
    out_ref[...] = acc * dis + b_ref[...]


def _final(degp, aggp, y, b2):
    return pl.pallas_call(
        _final_body,
        grid=(N_NODES // BF,),
        in_specs=[
            pl.BlockSpec((NC, BF, D), lambda i: (0, i, 0)),
            pl.BlockSpec((NC, BF, D), lambda i: (0, i, 0)),
            pl.BlockSpec((BF, D), lambda i: (i, 0)),
            pl.BlockSpec((1, D), lambda i: (0, 0)),
        ],
        out_specs=pl.BlockSpec((BF, D), lambda i: (i, 0)),
        out_shape=jax.ShapeDtypeStruct((N_NODES, D), jnp.float32),
    )(degp, aggp, y, b2)


def kernel(x, edge_index, W, b):
    ei = edge_index.astype(jnp.int32)
    pad = jnp.full((E_PAD - N_EDGES,), N_NODES, jnp.int32)
    src = jnp.concatenate([ei[0], pad]).reshape(EROWS, EB)
    dst = jnp.concatenate([ei[1], pad]).reshape(EROWS, EB)
    xp = jnp.pad(x, ((0, N_PAD - N_NODES), (0, 0)))
    onesD = jnp.ones((EB, D), jnp.float32)
    zerosD = jnp.zeros((N_PAD, D), jnp.float32)

    degp = _deg_kernel()(dst, onesD, zerosD)
    y = _mm(degp, xp, W)
    aggp = _agg_kernel()(src, dst, y, zerosD)
    out = _final(degp, aggp, y, b.reshape(1, D))
    return out


# speedup vs baseline: 15.2170x; 1.1280x over previous
import functools

import jax
import jax.numpy as jnp
from jax import lax
from jax.experimental import pallas as pl
from jax.experimental.pallas import tpu as pltpu
from jax.experimental.pallas import tpu_sc as plsc

N_NODES = 10000
N_EDGES = 320000
D = 128

NC = 2
NS = 16
NW = NC * NS

EB = 128
EROWS = 2560
E_PAD = EROWS * EB
ROWS_PER_W = EROWS // NW

N_PAD = 10240
NCH = N_PAD // NS


def _deg_body(dst_hbm, ones_hbm, zeros_hbm, degp_hbm, dstv, onesv, deg_sh):
    c = lax.axis_index("c")
    s = lax.axis_index("s")
    gid = c * NS + s
    row0 = gid * ROWS_PER_W
    pltpu.sync_copy(dst_hbm.at[pl.ds(row0, ROWS_PER_W)], dstv)
    pltpu.sync_copy(ones_hbm, onesv)
    n0 = s * NCH
    pltpu.sync_copy(zeros_hbm.at[pl.ds(n0, NCH)], deg_sh.at[pl.ds(n0, NCH)])
    plsc.subcore_barrier()

    def step(i, carry):
        pltpu.sync_copy(onesv, deg_sh.at[dstv.at[i]], add=True)
        return carry

    lax.fori_loop(0, ROWS_PER_W, step, 0)
    plsc.subcore_barrier()
    pltpu.sync_copy(deg_sh.at[pl.ds(n0, NCH)],
                    degp_hbm.at[c, pl.ds(n0, NCH)])


@functools.cache
def _deg_kernel():
    return pl.kernel(
        _deg_body,
        out_type=jax.ShapeDtypeStruct((NC, N_PAD, D), jnp.float32),
        mesh=plsc.VectorSubcoreMesh(core_axis_name="c", subcore_axis_name="s",
                                    num_cores=NC, num_subcores=NS),
        scratch_types=[
            pltpu.VMEM((ROWS_PER_W, EB), jnp.int32),
            pltpu.VMEM((EB, D), jnp.float32),
            pltpu.VMEM_SHARED((N_PAD, D), jnp.float32),
        ],
    )


NBUF = 2
IB = 40


def _agg_body(src_hbm, dst_hbm, y_hbm, zeros_hbm, aggp_hbm,
              srcv, dstv, rows, sem0, sem1, agg_sh):
    sems = (sem0, sem1)
    c = lax.axis_index("c")
    s = lax.axis_index("s")
    gid = c * NS + s
    row0 = gid * ROWS_PER_W
    n0 = s * NCH
    pltpu.sync_copy(zeros_hbm.at[pl.ds(n0, NCH)], agg_sh.at[pl.ds(n0, NCH)])
    plsc.subcore_barrier()

    for k in range(ROWS_PER_W // IB):
        pltpu.sync_copy(src_hbm.at[pl.ds(row0 + k * IB, IB)], srcv)
        pltpu.sync_copy(dst_hbm.at[pl.ds(row0 + k * IB, IB)], dstv)
        for b in range(NBUF):
            pltpu.async_copy(y_hbm.at[srcv.at[b]], rows.at[b], sems[b])

        def lap(i, carry):
            base = i * NBUF
            for b in range(NBUF):
                j = base + b
                pltpu.make_async_copy(y_hbm.at[srcv.at[j]], rows.at[b],
                                      sems[b]).wait()
                pltpu.sync_copy(rows.at[b], agg_sh.at[dstv.at[j]], add=True)
                nxt = j + NBUF

                @pl.when(nxt < IB)
                def _():
                    pltpu.async_copy(y_hbm.at[srcv.at[nxt]], rows.at[b],
                                     sems[b])
            return carry

        lax.fori_loop(0, IB // NBUF, lap, 0)

    plsc.subcore_barrier()
    pltpu.sync_copy(agg_sh.at[pl.ds(n0, NCH)],
                    aggp_hbm.at[c, pl.ds(n0, NCH)])


@functools.cache
def _agg_kernel():
    return pl.kernel(
        _agg_body,
        out_type=jax.ShapeDtypeStruct((NC, N_PAD, D), jnp.float32),
        mesh=plsc.VectorSubcoreMesh(core_axis_name="c", subcore_axis_name="s",
                                    num_cores=NC, num_subcores=NS),
        scratch_types=[
            pltpu.VMEM((IB, EB), jnp.int32),
            pltpu.VMEM((IB, EB), jnp.int32),
            pltpu.VMEM((NBUF, EB, D), jnp.float32),
            pltpu.SemaphoreType.DMA,
            pltpu.SemaphoreType.DMA,
            pltpu.VMEM_SHARED((N_PAD, D), jnp.float32),
        ],
    )


BM = 1024
BF = 1000


def _mm_body(degp_ref, x_ref, w_ref, y_ref):
    deg = degp_ref[0, :, 0:1] + degp_ref[1, :, 0:1] + 1.0
    dis = lax.rsqrt(deg)
    xw = jnp.dot(x_ref[...], w_ref[...], preferred_element_type=jnp.float32)
    y_ref[...] = xw * dis


def _mm(degp, xp, W):
    return pl.pallas_call(
        _mm_body,
        grid=(N_PAD // BM,),
        in_specs=[
            pl.BlockSpec((NC, BM, D), lambda i: (0, i, 0)),
            pl.BlockSpec((BM, D), lambda i: (i, 0)),
            pl.BlockSpec((D, D), lambda i: (0, 0)),
        ],
        out_specs=pl.BlockSpec((BM, D), lambda i: (i, 0)),
        out_shape=jax.ShapeDtypeStruct((N_PAD, D), jnp.float32),
    )(degp, xp, W)


def _final_body(degp_ref, aggp_ref, y_ref, b_ref, out_ref):
    deg = degp_ref[0, :, 0:1] + degp_ref[1, :, 0:1] + 1.0
    dis = lax.rsqrt(deg)
    acc = aggp_ref[0] + aggp_ref[1] + y_ref[...]
    out_ref[...] = acc * dis + b_ref[...]


def _final(degp, aggp, y, b2):
    return pl.pallas_call(
        _final_body,
        grid=(N_NODES // BF,),
        in_specs=[
            pl.BlockSpec((NC, BF, D), lambda i: (0, i, 0)),
            pl.BlockSpec((NC, BF, D), lambda i: (0, i, 0)),
            pl.BlockSpec((BF, D), lambda i: (i, 0)),
            pl.BlockSpec((1, D), lambda i: (0, 0)),
        ],
        out_specs=pl.BlockSpec((BF, D), lambda i: (i, 0)),
        out_shape=jax.ShapeDtypeStruct((N_NODES, D), jnp.float32),
    )(degp, aggp, y, b2)


def kernel(x, edge_index, W, b):
    ei = edge_index.astype(jnp.int32)
    pad = jnp.full((E_PAD - N_EDGES,), N_NODES, jnp.int32)
    src = jnp.concatenate([ei[0], pad]).reshape(EROWS, EB)
    dst = jnp.concatenate([ei[1], pad]).reshape(EROWS, EB)
    xp = jnp.pad(x, ((0, N_PAD - N_NODES), (0, 0)))
    onesD = jnp.ones((EB, D), jnp.float32)
    zerosD = jnp.zeros((N_PAD, D), jnp.float32)

    degp = _deg_kernel()(dst, onesD, zerosD)
    y = _mm(degp, xp, W)
    aggp = _agg_kernel()(src, dst, y, zerosD)
    out = _final(degp, aggp, y, b.reshape(1, D))
    return out
